# baseline (speedup 1.0000x reference)
import functools

import jax
import jax.numpy as jnp
from jax import lax
from jax.experimental import pallas as pl
from jax.experimental.pallas import tpu as pltpu
from jax.experimental.pallas import tpu_sc as plsc

_BUFFER_SIZE = 4194304
_TAIL = 1536
_BUF_LEN = _BUFFER_SIZE + _TAIL
_NC = 2
_NS = 16
_NW = _NC * _NS
_PER_W = _BUFFER_SIZE // _NW
_CH = 16384
_NCH = _PER_W // _CH

_MESH = plsc.VectorSubcoreMesh(core_axis_name="c", subcore_axis_name="s")


def _in_copy(x_hbm, stage, in_sems, base, c):
    return pltpu.make_async_copy(
        x_hbm.at[pl.ds(0, 1), pl.ds(base + c * _CH, _CH)],
        stage.at[pl.ds(c, 1), :],
        in_sems.at[c],
    )


def _out_copy(out_hbm, stage, out_sems, base, c):
    return pltpu.make_async_copy(
        stage.at[pl.ds(c, 1), :],
        out_hbm.at[pl.ds(0, 1), pl.ds(_TAIL + base + c * _CH, _CH)],
        out_sems.at[c],
    )


@functools.partial(
    pl.kernel,
    out_type=jax.ShapeDtypeStruct((1, _BUF_LEN), jnp.float32),
    mesh=_MESH,
    scratch_types=[
        pltpu.VMEM((_NCH, _CH), jnp.float32),
        pltpu.SemaphoreType.DMA((_NCH,)),
        pltpu.SemaphoreType.DMA((_NCH,)),
        pltpu.SemaphoreType.DMA,
        pltpu.SemaphoreType.DMA,
    ],
)
def _sc_concat(x_hbm, buf_hbm, out_hbm, stage, in_sems, out_sems,
               tin_sem, tout_sem):
    wid = lax.axis_index("s") * _NC + lax.axis_index("c")
    base = wid * _PER_W

    for c in range(_NCH):
        _in_copy(x_hbm, stage, in_sems, base, c).start()

    for c in range(_NCH):
        _in_copy(x_hbm, stage, in_sems, base, c).wait()
        _out_copy(out_hbm, stage, out_sems, base, c).start()

    _out_copy(out_hbm, stage, out_sems, base, 0).wait()

    @pl.when(wid == 0)
    def _():
        tslot = stage.at[pl.ds(0, 1), pl.ds(0, _TAIL)]
        t_in = pltpu.make_async_copy(
            buf_hbm.at[pl.ds(0, 1), pl.ds(_BUFFER_SIZE, _TAIL)], tslot,
            tin_sem)
        t_in.start()
        t_in.wait()
        pltpu.make_async_copy(
            tslot, out_hbm.at[pl.ds(0, 1), pl.ds(0, _TAIL)], tout_sem).start()

    for c in range(1, _NCH):
        _out_copy(out_hbm, stage, out_sems, base, c).wait()

    @pl.when(wid == 0)
    def _():
        pltpu.make_async_copy(
            stage.at[pl.ds(0, 1), pl.ds(0, _TAIL)],
            out_hbm.at[pl.ds(0, 1), pl.ds(0, _TAIL)], tout_sem).wait()


def kernel(x, buffer):
    return _sc_concat(x, buffer)


# speedup vs baseline: 1.0193x; 1.0193x over previous
import functools

import jax
import jax.numpy as jnp
from jax import lax
from jax.experimental import pallas as pl
from jax.experimental.pallas import tpu as pltpu
from jax.experimental.pallas import tpu_sc as plsc

_BUFFER_SIZE = 4194304
_TAIL = 1536
_BUF_LEN = _BUFFER_SIZE + _TAIL
_NC = 2
_NS = 16
_NW = _NC * _NS
_PER_W = _BUFFER_SIZE // _NW
_CHUNKS = (8192, 24576, 49152, 49152)
_OFFS = (0, 8192, 32768, 81920)
_NCH = len(_CHUNKS)

_MESH = plsc.VectorSubcoreMesh(core_axis_name="c", subcore_axis_name="s")


def _in_copy(x_hbm, stage, in_sems, base, c):
    return pltpu.make_async_copy(
        x_hbm.at[pl.ds(0, 1), pl.ds(base + _OFFS[c], _CHUNKS[c])],
        stage.at[pl.ds(0, 1), pl.ds(_OFFS[c], _CHUNKS[c])],
        in_sems.at[c],
    )


def _out_copy(out_hbm, stage, out_sems, base, c):
    return pltpu.make_async_copy(
        stage.at[pl.ds(0, 1), pl.ds(_OFFS[c], _CHUNKS[c])],
        out_hbm.at[pl.ds(0, 1), pl.ds(_TAIL + base + _OFFS[c], _CHUNKS[c])],
        out_sems.at[c],
    )


@functools.partial(
    pl.kernel,
    out_type=jax.ShapeDtypeStruct((1, _BUF_LEN), jnp.float32),
    mesh=_MESH,
    scratch_types=[
        pltpu.VMEM((1, _PER_W), jnp.float32),
        pltpu.SemaphoreType.DMA((_NCH,)),
        pltpu.SemaphoreType.DMA((_NCH,)),
        pltpu.SemaphoreType.DMA,
        pltpu.SemaphoreType.DMA,
    ],
)
def _sc_concat(x_hbm, buf_hbm, out_hbm, stage, in_sems, out_sems,
               tin_sem, tout_sem):
    wid = lax.axis_index("s") * _NC + lax.axis_index("c")
    base = wid * _PER_W

    for c in range(_NCH):
        _in_copy(x_hbm, stage, in_sems, base, c).start()

    for c in range(_NCH):
        _in_copy(x_hbm, stage, in_sems, base, c).wait()
        _out_copy(out_hbm, stage, out_sems, base, c).start()

    _out_copy(out_hbm, stage, out_sems, base, 0).wait()

    @pl.when(wid == 0)
    def _():
        tslot = stage.at[pl.ds(0, 1), pl.ds(0, _TAIL)]
        t_in = pltpu.make_async_copy(
            buf_hbm.at[pl.ds(0, 1), pl.ds(_BUFFER_SIZE, _TAIL)], tslot,
            tin_sem)
        t_in.start()
        t_in.wait()
        pltpu.make_async_copy(
            tslot, out_hbm.at[pl.ds(0, 1), pl.ds(0, _TAIL)], tout_sem).start()

    for c in range(1, _NCH):
        _out_copy(out_hbm, stage, out_sems, base, c).wait()

    @pl.when(wid == 0)
    def _():
        pltpu.make_async_copy(
            stage.at[pl.ds(0, 1), pl.ds(0, _TAIL)],
            out_hbm.at[pl.ds(0, 1), pl.ds(0, _TAIL)], tout_sem).wait()


def kernel(x, buffer):
    return _sc_concat(x, buffer)
